# baseline (speedup 1.0000x reference)
import functools

import jax
import jax.numpy as jnp
from jax import lax
from jax.experimental import pallas as pl
from jax.experimental.pallas import tpu as pltpu
from jax.experimental.pallas import tpu_sc as plsc

NEG = 0.2
N = 10000
NROWS = 10240
CH = 80
NTILES = 32
EP = 327680
BLK = 1000


def _sc_edge_pass(S, D, srcp, gdstp, sdstp, msg_w):
    MW = msg_w
    RW = MW + 16
    NCHK = EP // (NTILES * CH)
    RPT = NROWS // 16

    mesh = plsc.VectorSubcoreMesh(core_axis_name="c", subcore_axis_name="s")

    @functools.partial(
        pl.kernel, mesh=mesh,
        compiler_params=pltpu.CompilerParams(use_tc_tiling_on_sc=False),
        out_type=jax.ShapeDtypeStruct((2, NROWS, RW), jnp.float32),
        scratch_types=[
            pltpu.VMEM((CH,), jnp.int32),
            pltpu.VMEM((CH,), jnp.int32),
            pltpu.VMEM((CH,), jnp.int32),
            pltpu.VMEM((CH,), jnp.int32),
            pltpu.VMEM((CH,), jnp.int32),
            pltpu.VMEM((CH,), jnp.int32),
            pltpu.VMEM((CH, RW), jnp.float32),
            pltpu.VMEM((CH, 16), jnp.float32),
            pltpu.VMEM((CH, RW), jnp.float32),
            pltpu.VMEM((CH, 16), jnp.float32),
            pltpu.VMEM((CH, RW), jnp.float32),
            pltpu.VMEM_SHARED((NROWS, RW), jnp.float32),
            pltpu.SemaphoreType.DMA,
            pltpu.SemaphoreType.DMA,
            pltpu.SemaphoreType.DMA,
            pltpu.SemaphoreType.DMA,
            pltpu.SemaphoreType.DMA,
            pltpu.SemaphoreType.DMA,
        ],
    )
    def k(s_hbm, d_hbm, src_hbm, gdst_hbm, sdst_hbm, out_hbm,
          sidxa, gidxa, xidxa, sidxb, gidxb, xidxb,
          sbufa, dbufa, sbufb, dbufb, obuf,
          acc, semsa, semda, semsb, semdb, semia, semib):
        cid = lax.axis_index("c")
        sid = lax.axis_index("s")
        zero = jnp.zeros((16,), jnp.float32)

        def zrow(i, c):
            for g in range(RW // 16):
                obuf[i, pl.ds(g * 16, 16)] = zero
            return c
        lax.fori_loop(0, CH, zrow, 0)

        rbase = sid * RPT
        for t in range(RPT // CH):
            pltpu.sync_copy(obuf, acc.at[pl.ds(rbase + t * CH, CH)])
        plsc.subcore_barrier()

        wid = sid * 2 + cid
        ebase = wid * (NCHK * CH)

        def issue_idx(off, sidx, gidx, xidx, semi):
            pltpu.async_copy(src_hbm.at[pl.ds(off, CH)], sidx, semi)
            pltpu.async_copy(gdst_hbm.at[pl.ds(off, CH)], gidx, semi)
            pltpu.async_copy(sdst_hbm.at[pl.ds(off, CH)], xidx, semi)

        def wait_idx(sidx, gidx, xidx, semi):
            pltpu.make_async_copy(src_hbm.at[pl.ds(0, CH)], sidx, semi).wait()
            pltpu.make_async_copy(src_hbm.at[pl.ds(0, CH)], gidx, semi).wait()
            pltpu.make_async_copy(src_hbm.at[pl.ds(0, CH)], xidx, semi).wait()

        def issue_gather(sidx, gidx, sbuf, dbuf, sems, semd):
            pltpu.async_copy(s_hbm.at[sidx], sbuf, sems)
            pltpu.async_copy(d_hbm.at[gidx], dbuf, semd)

        def drain_gather(sbuf, dbuf, sems, semd):
            pltpu.make_async_copy(s_hbm.at[pl.ds(0, CH)], sbuf, sems).wait()
            pltpu.make_async_copy(d_hbm.at[pl.ds(0, CH)], dbuf, semd).wait()

        def compute(sbuf, dbuf, xidx):
            def edge(j, c2):
                u = sbuf[j, pl.ds(MW, 16)] + dbuf[j, pl.ds(0, 16)]
                u = jnp.where(u > 0.0, u, NEG * u)
                w16 = jnp.exp(u)
                obuf[j, pl.ds(MW, 16)] = w16
                for g in range(MW // 16):
                    obuf[j, pl.ds(g * 16, 16)] = (
                        sbuf[j, pl.ds(g * 16, 16)] * w16)
                return c2
            lax.fori_loop(0, CH, edge, 0)
            pltpu.sync_copy(obuf, acc.at[xidx], add=True)

        issue_idx(pl.multiple_of(ebase, 8), sidxa, gidxa, xidxa, semia)
        wait_idx(sidxa, gidxa, xidxa, semia)
        issue_gather(sidxa, gidxa, sbufa, dbufa, semsa, semda)
        issue_idx(pl.multiple_of(ebase + CH, 8), sidxb, gidxb, xidxb, semib)

        def pair(i2, c):
            wait_idx(sidxb, gidxb, xidxb, semib)
            issue_gather(sidxb, gidxb, sbufb, dbufb, semsb, semdb)
            offa = pl.multiple_of(ebase + (2 * i2 + 2) * CH, 8)
            drain_gather(sbufa, dbufa, semsa, semda)
            compute(sbufa, dbufa, xidxa)
            issue_idx(offa, sidxa, gidxa, xidxa, semia)
            wait_idx(sidxa, gidxa, xidxa, semia)
            issue_gather(sidxa, gidxa, sbufa, dbufa, semsa, semda)
            offb = pl.multiple_of(ebase + (2 * i2 + 3) * CH, 8)
            drain_gather(sbufb, dbufb, semsb, semdb)
            compute(sbufb, dbufb, xidxb)
            issue_idx(offb, sidxb, gidxb, xidxb, semib)
            return c
        lax.fori_loop(0, NCHK // 2, pair, 0)
        wait_idx(sidxb, gidxb, xidxb, semib)
        drain_gather(sbufa, dbufa, semsa, semda)

        plsc.subcore_barrier()
        pltpu.sync_copy(acc.at[pl.ds(rbase, RPT)],
                        out_hbm.at[cid, pl.ds(rbase, RPT)])

    return k(S, D, srcp, gdstp, sdstp)


def _tc1(x, W1, Q1):
    def body(x_ref, w_ref, q_ref, s_ref, d_ref):
        h = jnp.dot(x_ref[...], w_ref[...], preferred_element_type=jnp.float32)
        att = jnp.dot(h, q_ref[...], preferred_element_type=jnp.float32)
        asrc = att[:, 0:8]
        adst = att[:, 8:16]
        s_ref[...] = jnp.concatenate([h, asrc, asrc], axis=1)
        d_ref[...] = jnp.concatenate([adst, adst], axis=1)

    return pl.pallas_call(
        body,
        grid=(N // BLK,),
        in_specs=[
            pl.BlockSpec((BLK, 128), lambda i: (i, 0)),
            pl.BlockSpec((128, 128), lambda i: (0, 0)),
            pl.BlockSpec((128, 16), lambda i: (0, 0)),
        ],
        out_specs=[
            pl.BlockSpec((BLK, 144), lambda i: (i, 0)),
            pl.BlockSpec((BLK, 16), lambda i: (i, 0)),
        ],
        out_shape=[
            jax.ShapeDtypeStruct((N, 144), jnp.float32),
            jax.ShapeDtypeStruct((N, 16), jnp.float32),
        ],
    )(x, W1, Q1)


def _tc2(acc1, S1, D1, W2, Q2, PT, C):
    def body(a_ref, b_ref, s1_ref, d1_ref, w2_ref, q2_ref, pt_ref, c_ref,
             s2_ref, d2_ref):
        a = a_ref[0]
        b = b_ref[0]
        h1 = s1_ref[:, 0:128]
        asrc = s1_ref[:, 128:136]
        adst = d1_ref[:, 0:8]
        us = asrc + adst
        us = jnp.where(us > 0.0, us, NEG * us)
        ws = jnp.exp(us)
        den8 = a[:, 128:136] + b[:, 128:136] + ws
        wx = jnp.dot(ws, pt_ref[...], preferred_element_type=jnp.float32)
        dx = jnp.dot(den8, pt_ref[...], preferred_element_type=jnp.float32)
        msg = a[:, 0:128] + b[:, 0:128] + wx * h1
        cc = c_ref[...]
        g = msg / (dx + 1e-16) + cc[0:1, :]
        g = g * cc[1:2, :] + cc[2:3, :]
        g = jnp.where(g > 0.0, g, jnp.exp(g) - 1.0)
        h2 = jnp.dot(g, w2_ref[...], preferred_element_type=jnp.float32)
        att2 = jnp.dot(h2, q2_ref[...], preferred_element_type=jnp.float32)
        s2_ref[...] = jnp.concatenate([h2, att2[:, 0:16]], axis=1)
        d2_ref[...] = att2[:, 16:32]

    return pl.pallas_call(
        body,
        grid=(N // BLK,),
        in_specs=[
            pl.BlockSpec((1, BLK, 144), lambda i: (0, i, 0)),
            pl.BlockSpec((1, BLK, 144), lambda i: (1, i, 0)),
            pl.BlockSpec((BLK, 144), lambda i: (i, 0)),
            pl.BlockSpec((BLK, 16), lambda i: (i, 0)),
            pl.BlockSpec((128, 64), lambda i: (0, 0)),
            pl.BlockSpec((64, 32), lambda i: (0, 0)),
            pl.BlockSpec((8, 128), lambda i: (0, 0)),
            pl.BlockSpec((3, 128), lambda i: (0, 0)),
        ],
        out_specs=[
            pl.BlockSpec((BLK, 80), lambda i: (i, 0)),
            pl.BlockSpec((BLK, 16), lambda i: (i, 0)),
        ],
        out_shape=[
            jax.ShapeDtypeStruct((N, 80), jnp.float32),
            jax.ShapeDtypeStruct((N, 16), jnp.float32),
        ],
    )(acc1, acc1, S1, D1, W2, Q2, PT, C)


def _tc3(acc2, S2, D2, bias2):
    def body(a_ref, b_ref, s2_ref, d2_ref, b2_ref, o_ref):
        a = a_ref[0]
        b = b_ref[0]
        h2 = s2_ref[:, 0:64]
        u2 = s2_ref[:, 64:65] + d2_ref[:, 0:1]
        ws2 = jnp.exp(jnp.where(u2 > 0.0, u2, NEG * u2))
        den = a[:, 64:65] + b[:, 64:65] + ws2
        o = (a[:, 0:64] + b[:, 0:64] + ws2 * h2) / (den + 1e-16) + b2_ref[...]
        m = jnp.max(o, axis=1, keepdims=True)
        t = o - m
        lse = jnp.log(jnp.sum(jnp.exp(t), axis=1, keepdims=True))
        o_ref[...] = t - lse

    return pl.pallas_call(
        body,
        grid=(N // BLK,),
        in_specs=[
            pl.BlockSpec((1, BLK, 80), lambda i: (0, i, 0)),
            pl.BlockSpec((1, BLK, 80), lambda i: (1, i, 0)),
            pl.BlockSpec((BLK, 80), lambda i: (i, 0)),
            pl.BlockSpec((BLK, 16), lambda i: (i, 0)),
            pl.BlockSpec((1, 64), lambda i: (0, 0)),
        ],
        out_specs=pl.BlockSpec((BLK, 64), lambda i: (i, 0)),
        out_shape=jax.ShapeDtypeStruct((N, 64), jnp.float32),
    )(acc2, acc2, S2, D2, bias2)


def kernel(x, edge_index, W1, att_src1, att_dst1, bias1, bn_gamma, bn_beta,
           bn_mean, bn_var, W2, att_src2, att_dst2, bias2):
    f32 = jnp.float32
    src = edge_index[0].astype(jnp.int32)
    dst = edge_index[1].astype(jnp.int32)
    pad = EP + 2 * CH - src.shape[0]
    zpad = jnp.zeros((pad,), jnp.int32)
    srcp = jnp.concatenate([src, zpad])
    gdstp = jnp.concatenate([dst, zpad])
    sdstp = jnp.concatenate(
        [dst, N + jnp.arange(pad, dtype=jnp.int32) % (NROWS - N)])

    perm = jnp.asarray([(j % 8) * 16 + j // 8 for j in range(128)], jnp.int32)
    P8 = (jnp.arange(128)[:, None] // 16 == jnp.arange(8)[None, :]).astype(f32)
    a_s1 = att_src1.reshape(128)
    a_d1 = att_dst1.reshape(128)
    Q1 = jnp.concatenate([a_s1[:, None] * P8, a_d1[:, None] * P8], axis=1)
    Q1 = Q1[perm, :]
    W1p = W1[:, perm]
    a_s2 = att_src2.reshape(64)
    a_d2 = att_dst2.reshape(64)
    Q2 = jnp.concatenate([jnp.tile(a_s2[:, None], (1, 16)),
                          jnp.tile(a_d2[:, None], (1, 16))], axis=1)
    bn_s = bn_gamma / jnp.sqrt(bn_var + 1e-5)
    bn_b = bn_beta - bn_mean * bn_s
    C = jnp.stack([bias1[perm], bn_s[perm], bn_b[perm]])
    W2p = W2[perm, :]
    PT = (jnp.arange(128)[None, :] % 8 == jnp.arange(8)[:, None]).astype(f32)
    bias2r = bias2.reshape(1, 64)

    S1, D1 = _tc1(x, W1p, Q1)
    acc1 = _sc_edge_pass(S1, D1, srcp, gdstp, sdstp, 128)
    S2, D2 = _tc2(acc1, S1, D1, W2p, Q2, PT, C)
    acc2 = _sc_edge_pass(S2, D2, srcp, gdstp, sdstp, 64)
    return _tc3(acc2, S2, D2, bias2r)


# speedup vs baseline: 1.0396x; 1.0396x over previous
import functools

import jax
import jax.numpy as jnp
from jax import lax
from jax.experimental import pallas as pl
from jax.experimental.pallas import tpu as pltpu
from jax.experimental.pallas import tpu_sc as plsc

NEG = 0.2
N = 10000
NROWS = 10240
CH = 80
NTILES = 32
EP = 327680
BLK = 1000


def _sc_edge_pass(S, D, srcp, gdstp, sdstp, msg_w):
    MW = msg_w
    RW = MW + 16
    NCHK = EP // (NTILES * CH)
    RPT = NROWS // 16

    mesh = plsc.VectorSubcoreMesh(core_axis_name="c", subcore_axis_name="s")

    @functools.partial(
        pl.kernel, mesh=mesh,
        compiler_params=pltpu.CompilerParams(use_tc_tiling_on_sc=False),
        out_type=jax.ShapeDtypeStruct((2, NROWS, RW), jnp.float32),
        scratch_types=(
            [pltpu.VMEM((CH,), jnp.int32)] * 12
            + [
                pltpu.VMEM((CH, RW), jnp.float32),
                pltpu.VMEM((CH, 16), jnp.float32),
                pltpu.VMEM((CH, RW), jnp.float32),
                pltpu.VMEM((CH, 16), jnp.float32),
                pltpu.VMEM((CH, RW), jnp.float32),
                pltpu.VMEM_SHARED((NROWS, RW), jnp.float32),
            ]
            + [pltpu.SemaphoreType.DMA] * 8
        ),
    )
    def k(s_hbm, d_hbm, src_hbm, gdst_hbm, sdst_hbm, out_hbm,
          si0, gi0, xi0, si1, gi1, xi1, si2, gi2, xi2, si3, gi3, xi3,
          sbufa, dbufa, sbufb, dbufb, obuf, acc,
          semi0, semi1, semi2, semi3, semsa, semda, semsb, semdb):
        sidx = [si0, si1, si2, si3]
        gidx = [gi0, gi1, gi2, gi3]
        xidx = [xi0, xi1, xi2, xi3]
        semi = [semi0, semi1, semi2, semi3]
        sbufs = [sbufa, sbufb]
        dbufs = [dbufa, dbufb]
        semss = [semsa, semsb]
        semds = [semda, semdb]
        cid = lax.axis_index("c")
        sid = lax.axis_index("s")
        zero = jnp.zeros((16,), jnp.float32)

        def zrow(i, c):
            for g in range(RW // 16):
                obuf[i, pl.ds(g * 16, 16)] = zero
            return c
        lax.fori_loop(0, CH, zrow, 0)

        rbase = sid * RPT
        for t in range(RPT // CH):
            pltpu.sync_copy(obuf, acc.at[pl.ds(rbase + t * CH, CH)])
        plsc.subcore_barrier()

        wid = sid * 2 + cid
        ebase = wid * (NCHK * CH)

        def issue_idx(c, q):
            off = pl.multiple_of(ebase + c * CH, 8)
            pltpu.async_copy(src_hbm.at[pl.ds(off, CH)], sidx[q], semi[q])
            pltpu.async_copy(gdst_hbm.at[pl.ds(off, CH)], gidx[q], semi[q])
            pltpu.async_copy(sdst_hbm.at[pl.ds(off, CH)], xidx[q], semi[q])

        def wait_idx(q):
            s = semi[q]
            pltpu.make_async_copy(src_hbm.at[pl.ds(0, CH)], sidx[q], s).wait()
            pltpu.make_async_copy(src_hbm.at[pl.ds(0, CH)], gidx[q], s).wait()
            pltpu.make_async_copy(src_hbm.at[pl.ds(0, CH)], xidx[q], s).wait()

        def issue_gather(q, b):
            pltpu.async_copy(s_hbm.at[sidx[q]], sbufs[b], semss[b])
            pltpu.async_copy(d_hbm.at[gidx[q]], dbufs[b], semds[b])

        def drain_gather(b):
            pltpu.make_async_copy(s_hbm.at[pl.ds(0, CH)], sbufs[b],
                                  semss[b]).wait()
            pltpu.make_async_copy(d_hbm.at[pl.ds(0, CH)], dbufs[b],
                                  semds[b]).wait()

        def compute(b, q):
            sbuf = sbufs[b]
            dbuf = dbufs[b]

            def edge(j, c2):
                u = sbuf[j, pl.ds(MW, 16)] + dbuf[j, pl.ds(0, 16)]
                u = jnp.where(u > 0.0, u, NEG * u)
                w16 = jnp.exp(u)
                obuf[j, pl.ds(MW, 16)] = w16
                for g in range(MW // 16):
                    obuf[j, pl.ds(g * 16, 16)] = (
                        sbuf[j, pl.ds(g * 16, 16)] * w16)
                return c2
            lax.fori_loop(0, CH, edge, 0)
            pltpu.sync_copy(obuf, acc.at[xidx[q]], add=True)

        issue_idx(0, 0)
        wait_idx(0)
        issue_idx(1, 1)
        issue_gather(0, 0)
        wait_idx(1)

        def quad(i4, c):
            for k in range(4):
                qn = (k + 1) % 4
                qn2 = (k + 2) % 4
                issue_gather(qn, (k + 1) % 2)
                issue_idx(4 * i4 + k + 2, qn2)
                drain_gather(k % 2)
                compute(k % 2, k % 4)
                wait_idx(qn2)
            return c
        lax.fori_loop(0, NCHK // 4, quad, 0)
        drain_gather(0)

        plsc.subcore_barrier()
        pltpu.sync_copy(acc.at[pl.ds(rbase, RPT)],
                        out_hbm.at[cid, pl.ds(rbase, RPT)])

    return k(S, D, srcp, gdstp, sdstp)


def _tc1(x, W1, Q1):
    def body(x_ref, w_ref, q_ref, s_ref, d_ref):
        h = jnp.dot(x_ref[...], w_ref[...], preferred_element_type=jnp.float32)
        att = jnp.dot(h, q_ref[...], preferred_element_type=jnp.float32)
        asrc = att[:, 0:8]
        adst = att[:, 8:16]
        s_ref[...] = jnp.concatenate([h, asrc, asrc], axis=1)
        d_ref[...] = jnp.concatenate([adst, adst], axis=1)

    return pl.pallas_call(
        body,
        grid=(N // BLK,),
        in_specs=[
            pl.BlockSpec((BLK, 128), lambda i: (i, 0)),
            pl.BlockSpec((128, 128), lambda i: (0, 0)),
            pl.BlockSpec((128, 16), lambda i: (0, 0)),
        ],
        out_specs=[
            pl.BlockSpec((BLK, 144), lambda i: (i, 0)),
            pl.BlockSpec((BLK, 16), lambda i: (i, 0)),
        ],
        out_shape=[
            jax.ShapeDtypeStruct((N, 144), jnp.float32),
            jax.ShapeDtypeStruct((N, 16), jnp.float32),
        ],
    )(x, W1, Q1)


def _tc2(acc1, S1, D1, W2, Q2, PT, C):
    def body(a_ref, b_ref, s1_ref, d1_ref, w2_ref, q2_ref, pt_ref, c_ref,
             s2_ref, d2_ref):
        a = a_ref[0]
        b = b_ref[0]
        h1 = s1_ref[:, 0:128]
        asrc = s1_ref[:, 128:136]
        adst = d1_ref[:, 0:8]
        us = asrc + adst
        us = jnp.where(us > 0.0, us, NEG * us)
        ws = jnp.exp(us)
        den8 = a[:, 128:136] + b[:, 128:136] + ws
        wx = jnp.dot(ws, pt_ref[...], preferred_element_type=jnp.float32)
        dx = jnp.dot(den8, pt_ref[...], preferred_element_type=jnp.float32)
        msg = a[:, 0:128] + b[:, 0:128] + wx * h1
        cc = c_ref[...]
        g = msg / (dx + 1e-16) + cc[0:1, :]
        g = g * cc[1:2, :] + cc[2:3, :]
        g = jnp.where(g > 0.0, g, jnp.exp(g) - 1.0)
        h2 = jnp.dot(g, w2_ref[...], preferred_element_type=jnp.float32)
        att2 = jnp.dot(h2, q2_ref[...], preferred_element_type=jnp.float32)
        s2_ref[...] = jnp.concatenate([h2, att2[:, 0:16]], axis=1)
        d2_ref[...] = att2[:, 16:32]

    return pl.pallas_call(
        body,
        grid=(N // BLK,),
        in_specs=[
            pl.BlockSpec((1, BLK, 144), lambda i: (0, i, 0)),
            pl.BlockSpec((1, BLK, 144), lambda i: (1, i, 0)),
            pl.BlockSpec((BLK, 144), lambda i: (i, 0)),
            pl.BlockSpec((BLK, 16), lambda i: (i, 0)),
            pl.BlockSpec((128, 64), lambda i: (0, 0)),
            pl.BlockSpec((64, 32), lambda i: (0, 0)),
            pl.BlockSpec((8, 128), lambda i: (0, 0)),
            pl.BlockSpec((3, 128), lambda i: (0, 0)),
        ],
        out_specs=[
            pl.BlockSpec((BLK, 80), lambda i: (i, 0)),
            pl.BlockSpec((BLK, 16), lambda i: (i, 0)),
        ],
        out_shape=[
            jax.ShapeDtypeStruct((N, 80), jnp.float32),
            jax.ShapeDtypeStruct((N, 16), jnp.float32),
        ],
    )(acc1, acc1, S1, D1, W2, Q2, PT, C)


def _tc3(acc2, S2, D2, bias2):
    def body(a_ref, b_ref, s2_ref, d2_ref, b2_ref, o_ref):
        a = a_ref[0]
        b = b_ref[0]
        h2 = s2_ref[:, 0:64]
        u2 = s2_ref[:, 64:65] + d2_ref[:, 0:1]
        ws2 = jnp.exp(jnp.where(u2 > 0.0, u2, NEG * u2))
        den = a[:, 64:65] + b[:, 64:65] + ws2
        o = (a[:, 0:64] + b[:, 0:64] + ws2 * h2) / (den + 1e-16) + b2_ref[...]
        m = jnp.max(o, axis=1, keepdims=True)
        t = o - m
        lse = jnp.log(jnp.sum(jnp.exp(t), axis=1, keepdims=True))
        o_ref[...] = t - lse

    return pl.pallas_call(
        body,
        grid=(N // BLK,),
        in_specs=[
            pl.BlockSpec((1, BLK, 80), lambda i: (0, i, 0)),
            pl.BlockSpec((1, BLK, 80), lambda i: (1, i, 0)),
            pl.BlockSpec((BLK, 80), lambda i: (i, 0)),
            pl.BlockSpec((BLK, 16), lambda i: (i, 0)),
            pl.BlockSpec((1, 64), lambda i: (0, 0)),
        ],
        out_specs=pl.BlockSpec((BLK, 64), lambda i: (i, 0)),
        out_shape=jax.ShapeDtypeStruct((N, 64), jnp.float32),
    )(acc2, acc2, S2, D2, bias2)


def kernel(x, edge_index, W1, att_src1, att_dst1, bias1, bn_gamma, bn_beta,
           bn_mean, bn_var, W2, att_src2, att_dst2, bias2):
    f32 = jnp.float32
    src = edge_index[0].astype(jnp.int32)
    dst = edge_index[1].astype(jnp.int32)
    pad = EP + 2 * CH - src.shape[0]
    zpad = jnp.zeros((pad,), jnp.int32)
    srcp = jnp.concatenate([src, zpad])
    gdstp = jnp.concatenate([dst, zpad])
    sdstp = jnp.concatenate(
        [dst, N + jnp.arange(pad, dtype=jnp.int32) % (NROWS - N)])

    perm = jnp.asarray([(j % 8) * 16 + j // 8 for j in range(128)], jnp.int32)
    P8 = (jnp.arange(128)[:, None] // 16 == jnp.arange(8)[None, :]).astype(f32)
    a_s1 = att_src1.reshape(128)
    a_d1 = att_dst1.reshape(128)
    Q1 = jnp.concatenate([a_s1[:, None] * P8, a_d1[:, None] * P8], axis=1)
    Q1 = Q1[perm, :]
    W1p = W1[:, perm]
    a_s2 = att_src2.reshape(64)
    a_d2 = att_dst2.reshape(64)
    Q2 = jnp.concatenate([jnp.tile(a_s2[:, None], (1, 16)),
                          jnp.tile(a_d2[:, None], (1, 16))], axis=1)
    bn_s = bn_gamma / jnp.sqrt(bn_var + 1e-5)
    bn_b = bn_beta - bn_mean * bn_s
    C = jnp.stack([bias1[perm], bn_s[perm], bn_b[perm]])
    W2p = W2[perm, :]
    PT = (jnp.arange(128)[None, :] % 8 == jnp.arange(8)[:, None]).astype(f32)
    bias2r = bias2.reshape(1, 64)

    S1, D1 = _tc1(x, W1p, Q1)
    acc1 = _sc_edge_pass(S1, D1, srcp, gdstp, sdstp, 128)
    S2, D2 = _tc2(acc1, S1, D1, W2p, Q2, PT, C)
    acc2 = _sc_edge_pass(S2, D2, srcp, gdstp, sdstp, 64)
    return _tc3(acc2, S2, D2, bias2r)
